# Initial kernel scaffold; baseline (speedup 1.0000x reference)
import jax
import jax.numpy as jnp
from jax.experimental import pallas as pl


def kernel(input):
    raise NotImplementedError("write your pallas kernel here")


# speedup vs baseline: 32.9963x; 32.9963x over previous
import functools

import jax
import jax.numpy as jnp
from jax.experimental import pallas as pl
from jax.experimental.pallas import tpu as pltpu

B = 32
N = 4096
S = 1024


def _fps_body(x_ref, y_ref, z_ref, out_ref):
    x = x_ref[...]
    y = y_ref[...]
    z = z_ref[...]
    iota = jax.lax.broadcasted_iota(jnp.int32, (B, N), 1)
    lane_iota = jax.lax.broadcasted_iota(jnp.int32, (B, 128), 1)

    cx0 = x[:, 0:1]
    cy0 = y[:, 0:1]
    cz0 = z[:, 0:1]
    dists0 = jnp.full((B, N), jnp.inf, dtype=jnp.float32)

    def body(l, carry):
        dists, cx, cy, cz, buf = carry
        dx = x - cx
        dy = y - cy
        dz = z - cz
        d = (dx * dx + dy * dy) + dz * dz
        dists = jnp.minimum(dists, d)
        m = jnp.max(dists, axis=1, keepdims=True)
        idx = jnp.min(jnp.where(dists == m, iota, N), axis=1, keepdims=True)
        sel = iota == idx
        cx = jnp.sum(jnp.where(sel, x, 0.0), axis=1, keepdims=True)
        cy = jnp.sum(jnp.where(sel, y, 0.0), axis=1, keepdims=True)
        cz = jnp.sum(jnp.where(sel, z, 0.0), axis=1, keepdims=True)
        buf = jnp.where(lane_iota == l, idx, buf)
        return dists, cx, cy, cz, buf

    state = (dists0, cx0, cy0, cz0)
    for c in range(S // 128):
        buf0 = jnp.zeros((B, 128), jnp.int32)
        start = 1 if c == 0 else 0
        *state, buf = jax.lax.fori_loop(start, 128, body, (*state, buf0))
        out_ref[:, c * 128 : (c + 1) * 128] = buf
        state = tuple(state)


@jax.jit
def kernel(input):
    pts = input
    x = pts[:, :, 0]
    y = pts[:, :, 1]
    z = pts[:, :, 2]
    out = pl.pallas_call(
        _fps_body,
        out_shape=jax.ShapeDtypeStruct((B, S), jnp.int32),
    )(x, y, z)
    return out
